# Initial kernel scaffold; baseline (speedup 1.0000x reference)
import jax
import jax.numpy as jnp
from jax.experimental import pallas as pl


def kernel(main_out, gts):
    raise NotImplementedError("write your pallas kernel here")


# speedup vs baseline: 1.2371x; 1.2371x over previous
import jax
import jax.numpy as jnp
from jax.experimental import pallas as pl
from jax.experimental.pallas import tpu as pltpu

_T = 2.0
_C = 19


def _body(g_ref, xt_ref, out_ref, acc_s, acc_c, acc_l):
    i = pl.program_id(0)
    n = pl.num_programs(0)

    @pl.when(i == 0)
    def _init():
        acc_s[...] = jnp.zeros_like(acc_s)
        acc_c[...] = jnp.zeros_like(acc_c)
        acc_l[0] = 0.0

    x = xt_ref[...]
    g = g_ref[0]
    L = x.shape[1]
    cls = jax.lax.broadcasted_iota(jnp.int32, (_C, L), 0)
    oh = (g == cls).astype(jnp.float32)
    sums = jax.lax.dot_general(oh, x, (((1,), (1,)), ((), ())),
                               preferred_element_type=jnp.float32)
    cnt = jax.lax.dot_general(oh, jnp.ones((1, L), jnp.float32),
                              (((1,), (1,)), ((), ())),
                              preferred_element_type=jnp.float32)

    xs = x * (1.0 / _T)
    m = jnp.max(xs)
    e = jnp.exp(xs - m)
    s = jnp.sum(e, axis=0, keepdims=True)
    lse = jnp.log(s) + m

    acc_s[...] += sums
    acc_c[...] += cnt
    acc_l[0] += jnp.sum(lse)

    @pl.when(i == n - 1)
    def _fin():
        S = acc_s[...]
        Cn = acc_c[...]
        mean = S / jnp.maximum(Cn, 1.0)
        z = mean * (1.0 / _T)
        zm = jnp.max(z, axis=1, keepdims=True)
        ez = jnp.exp(z - zm)
        sz = jnp.sum(ez, axis=1, keepdims=True)
        q = ez / sz
        logq = (z - zm) - jnp.log(sz)
        term1 = jnp.sum(Cn * jnp.sum(q * logq, axis=1, keepdims=True))
        term2 = (1.0 / _T) * jnp.sum(q * S)
        kl = term1 - term2 + acc_l[0]
        out_ref[0] = kl


def kernel(main_out, gts):
    N = main_out.shape[0] * main_out.shape[1] * main_out.shape[2]
    L = 4096
    grid = N // L
    xt = main_out.reshape(N, _C).T
    gr = gts.reshape(-1).astype(jnp.int32).reshape(grid, 1, L)

    kl = pl.pallas_call(
        _body,
        grid=(grid,),
        in_specs=[
            pl.BlockSpec((1, 1, L), lambda i: (i, 0, 0)),
            pl.BlockSpec((_C, L), lambda i: (0, i)),
        ],
        out_specs=pl.BlockSpec(memory_space=pltpu.SMEM),
        out_shape=jax.ShapeDtypeStruct((1,), jnp.float32),
        scratch_shapes=[
            pltpu.VMEM((_C, _C), jnp.float32),
            pltpu.VMEM((_C, 1), jnp.float32),
            pltpu.SMEM((1,), jnp.float32),
        ],
    )(gr, xt)
    return kl[0] * (_T * _T / N)
